# Initial kernel scaffold; baseline (speedup 1.0000x reference)
import jax
import jax.numpy as jnp
from jax.experimental import pallas as pl


def kernel(v, e, edge_index, W1, b1, W2, b2):
    raise NotImplementedError("write your pallas kernel here")


# speedup vs baseline: 4.5283x; 4.5283x over previous
import functools

import jax
import jax.numpy as jnp
from jax import lax
from jax.experimental import pallas as pl
from jax.experimental.pallas import tpu as pltpu
from jax.experimental.pallas import tpu_sc as plsc

N_NODES = 10000
N_EDGES = 320000
HIDDEN = 128
CHUNK = 128
N_CHUNKS = N_EDGES // CHUNK
N_CORES = 2
N_SUBCORES = 16
CHUNKS_PER_CORE = N_CHUNKS // N_CORES
ROWS_MAIN = 624
TAIL_BASE = ROWS_MAIN * N_SUBCORES
TAIL_ROWS = N_NODES - TAIL_BASE

_MESH = plsc.VectorSubcoreMesh(core_axis_name="c", subcore_axis_name="s")


@functools.partial(
    pl.kernel,
    mesh=_MESH,
    out_type=jax.ShapeDtypeStruct((N_CORES, N_NODES, HIDDEN), jnp.float32),
    scratch_types=[
        pltpu.VMEM((CHUNK, HIDDEN), jnp.float32),
        pltpu.VMEM((1, CHUNK), jnp.int32),
        pltpu.VMEM_SHARED((N_NODES, HIDDEN), jnp.float32),
    ],
)
def _sc_segment_sum(e_hbm, idx_hbm, zeros_hbm, out_hbm, chunk_v, idx_v, acc):
    c = lax.axis_index("c")
    s = lax.axis_index("s")

    row0 = pl.multiple_of(s * ROWS_MAIN, 8)
    pltpu.sync_copy(zeros_hbm.at[pl.ds(0, ROWS_MAIN)],
                    acc.at[pl.ds(row0, ROWS_MAIN)])

    @pl.when(s == N_SUBCORES - 1)
    def _():
        pltpu.sync_copy(zeros_hbm.at[pl.ds(0, TAIL_ROWS)],
                        acc.at[pl.ds(TAIL_BASE, TAIL_ROWS)])

    plsc.subcore_barrier()

    n_chunks = 78 + jnp.where(s < 2, 1, 0)
    start = c * CHUNKS_PER_CORE + 78 * s + jnp.minimum(s, 2)

    def body(j, carry):
        chunk_id = start + j
        pltpu.sync_copy(idx_hbm.at[chunk_id], idx_v)
        base = pl.multiple_of(chunk_id * CHUNK, 8)
        pltpu.sync_copy(e_hbm.at[pl.ds(base, CHUNK)], chunk_v)
        pltpu.sync_copy(chunk_v, acc.at[idx_v.at[0]], add=True)
        return carry

    lax.fori_loop(0, n_chunks, body, 0)
    plsc.subcore_barrier()

    pltpu.sync_copy(acc.at[pl.ds(row0, ROWS_MAIN)],
                    out_hbm.at[c, pl.ds(row0, ROWS_MAIN)])

    @pl.when(s == N_SUBCORES - 1)
    def _():
        pltpu.sync_copy(acc.at[pl.ds(TAIL_BASE, TAIL_ROWS)],
                        out_hbm.at[c, pl.ds(TAIL_BASE, TAIL_ROWS)])


def _tc_mlp_body(part_ref, v_ref, w1t_ref, b1_ref, w2t_ref, b2_ref, out_ref):
    acc = part_ref[0] + part_ref[1]
    h = jnp.dot(acc, w1t_ref[...], preferred_element_type=jnp.float32)
    h = h + b1_ref[0]
    h = jnp.maximum(h, 0.0) + jnp.log1p(jnp.exp(-jnp.abs(h))) - 0.6931471805599453
    o = jnp.dot(h, w2t_ref[...], preferred_element_type=jnp.float32)
    out_ref[...] = o + b2_ref[0] + v_ref[...]


def _tc_mlp(partials, v, w1t, b1, w2t, b2):
    blk = 1000
    grid = (N_NODES // blk,)
    return pl.pallas_call(
        _tc_mlp_body,
        grid=grid,
        in_specs=[
            pl.BlockSpec((N_CORES, blk, HIDDEN), lambda i: (0, i, 0)),
            pl.BlockSpec((blk, HIDDEN), lambda i: (i, 0)),
            pl.BlockSpec((HIDDEN, HIDDEN), lambda i: (0, 0)),
            pl.BlockSpec((1, HIDDEN), lambda i: (0, 0)),
            pl.BlockSpec((HIDDEN, HIDDEN), lambda i: (0, 0)),
            pl.BlockSpec((1, HIDDEN), lambda i: (0, 0)),
        ],
        out_specs=pl.BlockSpec((blk, HIDDEN), lambda i: (i, 0)),
        out_shape=jax.ShapeDtypeStruct((N_NODES, HIDDEN), jnp.float32),
    )(partials, v, w1t, b1.reshape(1, HIDDEN), w2t, b2.reshape(1, HIDDEN))


def kernel(v, e, edge_index, W1, b1, W2, b2):
    dst = edge_index[1].reshape(N_CHUNKS, 1, CHUNK)
    zeros = jnp.zeros((ROWS_MAIN, HIDDEN), jnp.float32)
    partials = _sc_segment_sum(e, dst, zeros)
    return _tc_mlp(partials, v, W1.T, b1, W2.T, b2)


# speedup vs baseline: 7.8494x; 1.7334x over previous
import functools

import jax
import jax.numpy as jnp
from jax import lax
from jax.experimental import pallas as pl
from jax.experimental.pallas import tpu as pltpu
from jax.experimental.pallas import tpu_sc as plsc

N_NODES = 10000
N_EDGES = 320000
HIDDEN = 128
CHUNK = 128
N_CHUNKS = N_EDGES // CHUNK
N_CORES = 2
N_SUBCORES = 16
CHUNKS_PER_CORE = N_CHUNKS // N_CORES
ROWS_MAIN = 624
TAIL_BASE = ROWS_MAIN * N_SUBCORES
TAIL_ROWS = N_NODES - TAIL_BASE

_MESH = plsc.VectorSubcoreMesh(core_axis_name="c", subcore_axis_name="s")


@functools.partial(
    pl.kernel,
    mesh=_MESH,
    out_type=jax.ShapeDtypeStruct((N_CORES, N_NODES, HIDDEN), jnp.float32),
    scratch_types=[
        pltpu.VMEM((CHUNK, HIDDEN), jnp.float32),
        pltpu.VMEM((CHUNK, HIDDEN), jnp.float32),
        pltpu.VMEM((80, 1, CHUNK), jnp.int32),
        pltpu.VMEM_SHARED((N_NODES, HIDDEN), jnp.float32),
        pltpu.SemaphoreType.DMA,
        pltpu.SemaphoreType.DMA,
    ],
)
def _sc_segment_sum(e_hbm, idx_hbm, zeros_hbm, out_hbm, eb0, eb1, idx_v, acc,
                    sem0, sem1):
    c = lax.axis_index("c")
    s = lax.axis_index("s")

    n_e = 78 + jnp.where(s == 0, 2, 0)
    start = c * CHUNKS_PER_CORE + 78 * s + 2 * jnp.minimum(s, 1)

    idx_cp = pltpu.async_copy(idx_hbm.at[pl.ds(start, 80)], idx_v, sem0)

    row0 = pl.multiple_of(s * ROWS_MAIN, 8)
    pltpu.sync_copy(zeros_hbm.at[pl.ds(0, ROWS_MAIN)],
                    acc.at[pl.ds(row0, ROWS_MAIN)])

    @pl.when(s == N_SUBCORES - 1)
    def _():
        pltpu.sync_copy(zeros_hbm.at[pl.ds(0, TAIL_ROWS)],
                        acc.at[pl.ds(TAIL_BASE, TAIL_ROWS)])

    idx_cp.wait()
    plsc.subcore_barrier()

    def e_start(j, buf, sem):
        base = pl.multiple_of((start + j) * CHUNK, 8)
        return pltpu.async_copy(e_hbm.at[pl.ds(base, CHUNK)], buf, sem)

    def scatter(j, buf):
        pltpu.sync_copy(buf, acc.at[idx_v.at[j].at[0]], add=True)

    e_start(0, eb0, sem0)

    def body(jj, carry):
        j0 = 2 * jj

        @pl.when(j0 < n_e)
        def _():
            @pl.when(j0 + 1 < n_e)
            def _():
                e_start(j0 + 1, eb1, sem1)
            pltpu.make_async_copy(e_hbm.at[pl.ds(0, CHUNK)], eb0, sem0).wait()
            scatter(j0, eb0)

        j1 = j0 + 1

        @pl.when(j1 < n_e)
        def _():
            @pl.when(j1 + 1 < n_e)
            def _():
                e_start(j1 + 1, eb0, sem0)
            pltpu.make_async_copy(e_hbm.at[pl.ds(0, CHUNK)], eb1, sem1).wait()
            scatter(j1, eb1)

        return carry

    lax.fori_loop(0, 40, body, 0)
    plsc.subcore_barrier()

    pltpu.sync_copy(acc.at[pl.ds(row0, ROWS_MAIN)],
                    out_hbm.at[c, pl.ds(row0, ROWS_MAIN)])

    @pl.when(s == N_SUBCORES - 1)
    def _():
        pltpu.sync_copy(acc.at[pl.ds(TAIL_BASE, TAIL_ROWS)],
                        out_hbm.at[c, pl.ds(TAIL_BASE, TAIL_ROWS)])


def _tc_mlp_body(part_ref, v_ref, w1t_ref, b1_ref, w2t_ref, b2_ref, out_ref):
    acc = part_ref[0] + part_ref[1]
    h = jnp.dot(acc, w1t_ref[...], preferred_element_type=jnp.float32)
    h = h + b1_ref[0]
    h = jnp.maximum(h, 0.0) + jnp.log1p(jnp.exp(-jnp.abs(h))) - 0.6931471805599453
    o = jnp.dot(h, w2t_ref[...], preferred_element_type=jnp.float32)
    out_ref[...] = o + b2_ref[0] + v_ref[...]


def _tc_mlp(partials, v, w1t, b1, w2t, b2):
    blk = 1000
    grid = (N_NODES // blk,)
    return pl.pallas_call(
        _tc_mlp_body,
        grid=grid,
        in_specs=[
            pl.BlockSpec((N_CORES, blk, HIDDEN), lambda i: (0, i, 0)),
            pl.BlockSpec((blk, HIDDEN), lambda i: (i, 0)),
            pl.BlockSpec((HIDDEN, HIDDEN), lambda i: (0, 0)),
            pl.BlockSpec((1, HIDDEN), lambda i: (0, 0)),
            pl.BlockSpec((HIDDEN, HIDDEN), lambda i: (0, 0)),
            pl.BlockSpec((1, HIDDEN), lambda i: (0, 0)),
        ],
        out_specs=pl.BlockSpec((blk, HIDDEN), lambda i: (i, 0)),
        out_shape=jax.ShapeDtypeStruct((N_NODES, HIDDEN), jnp.float32),
    )(partials, v, w1t, b1.reshape(1, HIDDEN), w2t, b2.reshape(1, HIDDEN))


def kernel(v, e, edge_index, W1, b1, W2, b2):
    dst = edge_index[1].reshape(N_CHUNKS, 1, CHUNK)
    dst = jnp.concatenate(
        [dst, jnp.zeros((60, 1, CHUNK), jnp.int32)], axis=0)
    zeros = jnp.zeros((ROWS_MAIN, HIDDEN), jnp.float32)
    partials = _sc_segment_sum(e, dst, zeros)
    return _tc_mlp(partials, v, W1.T, b1, W2.T, b2)
